# baseline (speedup 1.0000x reference)
import functools

import jax
import jax.numpy as jnp
from jax import lax
from jax.experimental import pallas as pl
from jax.experimental.pallas import tpu as pltpu
from jax.experimental.pallas import tpu_sc as plsc

N = 10000
E = 320000
D = 128

NS = 16

CHUNK = 128
CH_PER_W = 160
EP = NS * CH_PER_W * CHUNK
NPAD = 10112
ROWS_PER_S = NPAD // NS

BLK = 8
NBLK = CH_PER_W // BLK
GPB = BLK // 2


def _mm_body(feat_ref, w_ref, b_ref, o_ref):
  o_ref[...] = (
      jnp.dot(feat_ref[...], w_ref[...], preferred_element_type=jnp.float32)
      + b_ref[...]
  )


_sc_mesh = plsc.VectorSubcoreMesh(
    core_axis_name="c", subcore_axis_name="s", num_cores=1)


@functools.partial(
    pl.kernel,
    out_type=jax.ShapeDtypeStruct((NPAD, D), jnp.float32),
    mesh=_sc_mesh,
    scratch_types=[
        pltpu.VMEM((2, BLK, CHUNK), jnp.int32),
        pltpu.VMEM((2, BLK, CHUNK), jnp.int32),
        pltpu.VMEM((2, CHUNK, D), jnp.float32),
        pltpu.VMEM_SHARED((NPAD, D), jnp.float32),
        pltpu.SemaphoreType.DMA((2,)),
        pltpu.SemaphoreType.DMA((2,)),
        pltpu.SemaphoreType.DMA,
    ],
)
def _sc_aggregate(src_hbm, dst_hbm, h_hbm, z_hbm, out_hbm,
                  sidx, didx, rows_v, acc, gsem, ssem, isem):
  s = lax.axis_index("s")
  row0 = s * CH_PER_W

  pltpu.sync_copy(z_hbm.at[pl.ds(s * ROWS_PER_S, ROWS_PER_S)],
                  acc.at[pl.ds(s * ROWS_PER_S, ROWS_PER_S)])
  pltpu.sync_copy(src_hbm.at[pl.ds(row0, BLK)], sidx.at[0])
  pltpu.sync_copy(dst_hbm.at[pl.ds(row0, BLK)], didx.at[0])
  plsc.subcore_barrier()

  def fire_idx(blk, slot):
    off = row0 + blk * BLK
    pltpu.async_copy(src_hbm.at[pl.ds(off, BLK)], sidx.at[slot], isem)
    pltpu.async_copy(dst_hbm.at[pl.ds(off, BLK)], didx.at[slot], isem)

  def drain_idx():
    pltpu.make_async_copy(src_hbm.at[pl.ds(0, BLK)], sidx.at[0], isem).wait()
    pltpu.make_async_copy(src_hbm.at[pl.ds(0, BLK)], didx.at[0], isem).wait()

  def fire_gather(slot, row, b):
    pltpu.async_copy(h_hbm.at[sidx.at[slot, row]], rows_v.at[b], gsem.at[b])

  def drain_gather(b):
    pltpu.make_async_copy(h_hbm.at[pl.ds(0, CHUNK)], rows_v.at[b],
                          gsem.at[b]).wait()

  def fire_scatter(slot, row, b):
    pltpu.async_copy(rows_v.at[b], acc.at[didx.at[slot, row]], ssem.at[b],
                     add=True)

  def drain_scatter(b):
    pltpu.make_async_copy(h_hbm.at[pl.ds(0, CHUNK)], rows_v.at[b],
                          ssem.at[b]).wait()

  fire_idx(1, 1)
  fire_gather(0, 0, 0)
  fire_gather(0, 1, 1)

  def block_body(blk, p, fire_next, fire_idx_next):
    q = 1 - p
    for gg in range(GPB):
      r0 = 2 * gg
      drain_gather(0)
      fire_scatter(p, r0, 0)
      drain_gather(1)
      fire_scatter(p, r0 + 1, 1)
      if gg == GPB - 1 and (fire_next or fire_idx_next):
        drain_idx()
      drain_scatter(0)
      drain_scatter(1)
      if gg < GPB - 1:
        fire_gather(p, r0 + 2, 0)
        fire_gather(p, r0 + 3, 1)
      elif fire_next:
        fire_gather(q, 0, 0)
        fire_gather(q, 1, 1)
    if fire_idx_next:
      fire_idx(blk + 2, p)

  def body(blk, carry):
    p = lax.rem(blk, 2)

    @pl.when(blk < NBLK - 2)
    def _steady():
      block_body(blk, p, True, True)

    @pl.when(blk == NBLK - 2)
    def _penultimate():
      block_body(blk, p, True, False)

    return carry

  lax.fori_loop(0, NBLK - 1, body, 0)
  block_body(NBLK - 1, (NBLK - 1) % 2, False, False)

  plsc.subcore_barrier()
  pltpu.sync_copy(acc.at[pl.ds(s * ROWS_PER_S, ROWS_PER_S)],
                  out_hbm.at[pl.ds(s * ROWS_PER_S, ROWS_PER_S)])


def kernel(feat, edge_index, W, b):
  src = edge_index[0].astype(jnp.int32)
  dst = edge_index[1].astype(jnp.int32)
  pad = EP - E
  srcp = jnp.concatenate([src, jnp.zeros((pad,), jnp.int32)]).reshape(-1, CHUNK)
  dstp = jnp.concatenate([dst, jnp.full((pad,), N, jnp.int32)]).reshape(-1, CHUNK)

  h = pl.pallas_call(
      _mm_body,
      grid=(10,),
      in_specs=[
          pl.BlockSpec((N // 10, D), lambda i: (i, 0)),
          pl.BlockSpec((D, D), lambda i: (0, 0)),
          pl.BlockSpec((1, D), lambda i: (0, 0)),
      ],
      out_specs=pl.BlockSpec((N // 10, D), lambda i: (i, 0)),
      out_shape=jax.ShapeDtypeStruct((N, D), jnp.float32),
  )(feat, W, b.reshape(1, D))

  zeros = jnp.zeros((NPAD, D), jnp.float32)
  out = _sc_aggregate(srcp, dstp, h, zeros)
  return out[:N]


# speedup vs baseline: 1.2497x; 1.2497x over previous
import functools

import jax
import jax.numpy as jnp
from jax import lax
from jax.experimental import pallas as pl
from jax.experimental.pallas import tpu as pltpu
from jax.experimental.pallas import tpu_sc as plsc

N = 10000
E = 320000
D = 128

NC = 2
NS = 16
NW = NC * NS

CHUNK = 128
CH_PER_W = 80
EP = NW * CH_PER_W * CHUNK
NPAD = 10112
ROWS_PER_S = NPAD // NS

BLK = 8
NBLK = CH_PER_W // BLK
NBUF = 2
GPB = BLK // NBUF


def _mm_body(feat_ref, w_ref, b_ref, o_ref):
  o_ref[...] = (
      jnp.dot(feat_ref[...], w_ref[...], preferred_element_type=jnp.float32)
      + b_ref[...]
  )


def _add_body(a_ref, b_ref, o_ref):
  o_ref[...] = a_ref[...] + b_ref[...]


_sc_mesh = plsc.VectorSubcoreMesh(core_axis_name="c", subcore_axis_name="s")


@functools.partial(
    pl.kernel,
    out_type=jax.ShapeDtypeStruct((NC, NPAD, D), jnp.float32),
    mesh=_sc_mesh,
    scratch_types=[
        pltpu.VMEM((2, BLK, CHUNK), jnp.int32),
        pltpu.VMEM((2, BLK, CHUNK), jnp.int32),
        pltpu.VMEM((NBUF, CHUNK, D), jnp.float32),
        pltpu.VMEM_SHARED((NPAD, D), jnp.float32),
        pltpu.SemaphoreType.DMA((NBUF,)),
        pltpu.SemaphoreType.DMA((NBUF,)),
        pltpu.SemaphoreType.DMA,
    ],
)
def _sc_aggregate(src_hbm, dst_hbm, h_hbm, z_hbm, out_hbm,
                  sidx, didx, rows_v, acc, gsem, ssem, isem):
  c = lax.axis_index("c")
  s = lax.axis_index("s")
  wid = c * NS + s
  row0 = wid * CH_PER_W

  pltpu.sync_copy(z_hbm.at[pl.ds(s * ROWS_PER_S, ROWS_PER_S)],
                  acc.at[pl.ds(s * ROWS_PER_S, ROWS_PER_S)])
  pltpu.sync_copy(src_hbm.at[pl.ds(row0, BLK)], sidx.at[0])
  pltpu.sync_copy(dst_hbm.at[pl.ds(row0, BLK)], didx.at[0])
  plsc.subcore_barrier()

  def fire_idx(blk, slot):
    off = row0 + blk * BLK
    pltpu.async_copy(src_hbm.at[pl.ds(off, BLK)], sidx.at[slot], isem)
    pltpu.async_copy(dst_hbm.at[pl.ds(off, BLK)], didx.at[slot], isem)

  def drain_idx():
    pltpu.make_async_copy(src_hbm.at[pl.ds(0, BLK)], sidx.at[0], isem).wait()
    pltpu.make_async_copy(src_hbm.at[pl.ds(0, BLK)], didx.at[0], isem).wait()

  def fire_gather(slot, row, b):
    pltpu.async_copy(h_hbm.at[sidx.at[slot, row]], rows_v.at[b], gsem.at[b])

  def drain_gather(b):
    pltpu.make_async_copy(h_hbm.at[pl.ds(0, CHUNK)], rows_v.at[b],
                          gsem.at[b]).wait()

  def fire_scatter(slot, row, b):
    pltpu.async_copy(rows_v.at[b], acc.at[didx.at[slot, row]], ssem.at[b],
                     add=True)

  def drain_scatter(b):
    pltpu.make_async_copy(h_hbm.at[pl.ds(0, CHUNK)], rows_v.at[b],
                          ssem.at[b]).wait()

  fire_idx(1, 1)
  for b in range(NBUF):
    fire_gather(0, b, b)

  def block_body(blk, p, fire_next, fire_idx_next):
    q = 1 - p
    for gg in range(GPB):
      r0 = NBUF * gg
      for b in range(NBUF):
        drain_gather(b)
        fire_scatter(p, r0 + b, b)
      if gg == GPB - 1 and (fire_next or fire_idx_next):
        drain_idx()
      for b in range(NBUF):
        drain_scatter(b)
        if gg < GPB - 1:
          fire_gather(p, r0 + NBUF + b, b)
        elif fire_next:
          fire_gather(q, b, b)
    if fire_idx_next:
      fire_idx(blk + 2, p)

  def body(blk, carry):
    p = lax.rem(blk, 2)

    @pl.when(blk < NBLK - 2)
    def _steady():
      block_body(blk, p, True, True)

    @pl.when(blk == NBLK - 2)
    def _penultimate():
      block_body(blk, p, True, False)

    return carry

  lax.fori_loop(0, NBLK - 1, body, 0)
  block_body(NBLK - 1, (NBLK - 1) % 2, False, False)

  plsc.subcore_barrier()
  pltpu.sync_copy(acc.at[pl.ds(s * ROWS_PER_S, ROWS_PER_S)],
                  out_hbm.at[c, pl.ds(s * ROWS_PER_S, ROWS_PER_S)])


def kernel(feat, edge_index, W, b):
  src = edge_index[0].astype(jnp.int32)
  dst = edge_index[1].astype(jnp.int32)
  pad = EP - E
  srcp = jnp.concatenate([src, jnp.zeros((pad,), jnp.int32)]).reshape(-1, CHUNK)
  dstp = jnp.concatenate([dst, jnp.full((pad,), N, jnp.int32)]).reshape(-1, CHUNK)

  h = pl.pallas_call(
      _mm_body,
      grid=(10,),
      in_specs=[
          pl.BlockSpec((N // 10, D), lambda i: (i, 0)),
          pl.BlockSpec((D, D), lambda i: (0, 0)),
          pl.BlockSpec((1, D), lambda i: (0, 0)),
      ],
      out_specs=pl.BlockSpec((N // 10, D), lambda i: (i, 0)),
      out_shape=jax.ShapeDtypeStruct((N, D), jnp.float32),
  )(feat, W, b.reshape(1, D))

  zeros = jnp.zeros((NPAD, D), jnp.float32)
  partials = _sc_aggregate(srcp, dstp, h, zeros)

  out = pl.pallas_call(
      _add_body,
      grid=(10,),
      in_specs=[
          pl.BlockSpec((N // 10, D), lambda i: (i, 0)),
          pl.BlockSpec((N // 10, D), lambda i: (i, 0)),
      ],
      out_specs=pl.BlockSpec((N // 10, D), lambda i: (i, 0)),
      out_shape=jax.ShapeDtypeStruct((N, D), jnp.float32),
  )(partials[0, :N], partials[1, :N])
  return out
